# Initial kernel scaffold; baseline (speedup 1.0000x reference)
import jax
import jax.numpy as jnp
from jax.experimental import pallas as pl


def kernel(x, edge_index, w1, b1, w2, b2, w3, b3, gw, gb):
    raise NotImplementedError("write your pallas kernel here")


# speedup vs baseline: 11.2376x; 11.2376x over previous
import functools

import jax
import jax.numpy as jnp
from jax import lax
from jax.experimental import pallas as pl
from jax.experimental.pallas import tpu as pltpu
from jax.experimental.pallas import tpu_sc as plsc

N = 10000
T = 12
C_IN = 128
HID = 256
C_OUT = 128
K = 3
TP = T - K + 1

NC = 2
NS = 16
L = 16
NW = NC * NS
CHUNK = 128
DW = 128
NPAD = 10112
STRIPE = NPAD // NS

NB = 1000
GRID = N // NB

def _sc_mesh():
    return plsc.VectorSubcoreMesh(
        core_axis_name="c", subcore_axis_name="s", num_cores=NC, num_subcores=NS)


def _deg_body(nchunks, dst_hbm, out_hbm, dst1_v, ones_v, deg_sp):
    c = lax.axis_index("c")
    s = lax.axis_index("s")
    wid = c * NS + s
    ept = nchunks * CHUNK
    one = jnp.full((L,), 1.0, jnp.float32)
    zero = jnp.zeros((L,), jnp.float32)

    def fill_zero(i, _):
        for k in range(DW // L):
            ones_v[i, L * k:L * (k + 1)] = zero
        return 0

    lax.fori_loop(0, CHUNK, fill_zero, 0)
    base = s * STRIPE
    for z in range(STRIPE // CHUNK):
        pltpu.sync_copy(ones_v, deg_sp.at[pl.ds(base + z * CHUNK, CHUNK)])
    rem = STRIPE % CHUNK
    if rem:
        pltpu.sync_copy(ones_v.at[pl.ds(0, rem)],
                        deg_sp.at[pl.ds(base + (STRIPE // CHUNK) * CHUNK, rem)])

    def fill_ones(i, _):
        for k in range(DW // L):
            ones_v[i, L * k:L * (k + 1)] = one
        return 0

    lax.fori_loop(0, CHUNK, fill_ones, 0)
    plsc.subcore_barrier()

    def body(j, _):
        pltpu.sync_copy(dst_hbm.at[pl.ds(wid * ept + j * CHUNK, CHUNK)], dst1_v)
        pltpu.sync_copy(ones_v, deg_sp.at[dst1_v], add=True)
        return 0

    lax.fori_loop(0, nchunks, body, 0)
    plsc.subcore_barrier()
    pltpu.sync_copy(deg_sp.at[pl.ds(s * STRIPE, STRIPE)],
                    out_hbm.at[c, pl.ds(s * STRIPE, STRIPE)])


def _scatter_body(nchunks, m_hbm, src_hbm, dst_hbm, out_hbm,
                  src1_v, dst1_v, rows_v, acc_sp):
    c = lax.axis_index("c")
    s = lax.axis_index("s")
    wid = c * NS + s
    ept = nchunks * CHUNK
    zero = jnp.zeros((L,), jnp.float32)

    def fill_zero(i, _):
        for k in range(C_OUT // L):
            rows_v[i, L * k:L * (k + 1)] = zero
        return 0

    lax.fori_loop(0, CHUNK, fill_zero, 0)
    base = s * STRIPE
    for z in range(STRIPE // CHUNK):
        pltpu.sync_copy(rows_v, acc_sp.at[pl.ds(base + z * CHUNK, CHUNK)])
    rem = STRIPE % CHUNK
    if rem:
        pltpu.sync_copy(rows_v.at[pl.ds(0, rem)],
                        acc_sp.at[pl.ds(base + (STRIPE // CHUNK) * CHUNK, rem)])
    plsc.subcore_barrier()

    def body(j, _):
        pltpu.sync_copy(src_hbm.at[pl.ds(wid * ept + j * CHUNK, CHUNK)], src1_v)
        pltpu.sync_copy(dst_hbm.at[pl.ds(wid * ept + j * CHUNK, CHUNK)], dst1_v)
        pltpu.sync_copy(m_hbm.at[src1_v], rows_v)
        pltpu.sync_copy(rows_v, acc_sp.at[dst1_v], add=True)
        return 0

    lax.fori_loop(0, nchunks, body, 0)
    plsc.subcore_barrier()
    pltpu.sync_copy(acc_sp.at[pl.ds(s * STRIPE, STRIPE)],
                    out_hbm.at[c, pl.ds(s * STRIPE, STRIPE)])


def _dense_body(x_ref, w1_ref, w2_ref, w3_ref, b_ref, gw_ref, degp_ref, m_ref):
    xb = x_ref[...]
    b1 = b_ref[0:1, :]
    b2 = b_ref[1:2, :]
    b3 = b_ref[2:3, :]
    w1 = w1_ref[...]
    w2 = w2_ref[...]
    w3 = w3_ref[...]
    y = [xb[:, t, :] for t in range(T)]
    acc = jnp.zeros((NB, HID), jnp.float32)
    for t in range(TP):
        xw = jnp.concatenate([y[t], y[t + 1], y[t + 2]], axis=1)
        p = jnp.dot(xw, w1, preferred_element_type=jnp.float32) + b1
        q = jnp.dot(xw, w2, preferred_element_type=jnp.float32) + b2
        r = jnp.dot(xw, w3, preferred_element_type=jnp.float32) + b3
        acc = acc + jnp.maximum(p * jax.nn.sigmoid(q) + r, 0.0)
    h = acc * jnp.float32(1.0 / TP)
    h2 = jnp.dot(h, gw_ref[...], preferred_element_type=jnp.float32)
    deg = degp_ref[0, :, 0:1] + degp_ref[1, :, 0:1] + 1.0
    m_ref[...] = h2 * lax.rsqrt(deg)


def _final_body(accp_ref, m_ref, degp_ref, gb_ref, out_ref):
    deg = degp_ref[0, :, 0:1] + degp_ref[1, :, 0:1] + 1.0
    dinv = lax.rsqrt(deg)
    tot = accp_ref[0] + accp_ref[1] + m_ref[...]
    out_ref[...] = jnp.maximum(tot * dinv + gb_ref[...], 0.0)


def _make_deg_call(nchunks):
    return pl.kernel(
        functools.partial(_deg_body, nchunks),
        out_type=jax.ShapeDtypeStruct((NC, NPAD, DW), jnp.float32),
        mesh=_sc_mesh(),
        scratch_types=[
            pltpu.VMEM((CHUNK,), jnp.int32),
            pltpu.VMEM((CHUNK, DW), jnp.float32),
            pltpu.VMEM_SHARED((NPAD, DW), jnp.float32),
        ],
    )


def _make_scatter_call(nchunks):
    return pl.kernel(
        functools.partial(_scatter_body, nchunks),
        out_type=jax.ShapeDtypeStruct((NC, NPAD, C_OUT), jnp.float32),
        mesh=_sc_mesh(),
        scratch_types=[
            pltpu.VMEM((CHUNK,), jnp.int32),
            pltpu.VMEM((CHUNK,), jnp.int32),
            pltpu.VMEM((CHUNK, C_OUT), jnp.float32),
            pltpu.VMEM_SHARED((NPAD, C_OUT), jnp.float32),
        ],
    )


def _dense_call(x, W1, W2, W3, bstack, gw, degp):
    return pl.pallas_call(
        _dense_body,
        grid=(GRID,),
        in_specs=[
            pl.BlockSpec((NB, T, C_IN), lambda i: (i, 0, 0)),
            pl.BlockSpec((K * C_IN, HID), lambda i: (0, 0)),
            pl.BlockSpec((K * C_IN, HID), lambda i: (0, 0)),
            pl.BlockSpec((K * C_IN, HID), lambda i: (0, 0)),
            pl.BlockSpec((4, HID), lambda i: (0, 0)),
            pl.BlockSpec((HID, C_OUT), lambda i: (0, 0)),
            pl.BlockSpec((NC, NB, DW), lambda i: (0, i, 0)),
        ],
        out_specs=pl.BlockSpec((NB, C_OUT), lambda i: (i, 0)),
        out_shape=jax.ShapeDtypeStruct((N, C_OUT), jnp.float32),
    )(x, W1, W2, W3, bstack, gw, degp)


def _final_call(accp, m, degp, gb2):
    return pl.pallas_call(
        _final_body,
        grid=(GRID,),
        in_specs=[
            pl.BlockSpec((NC, NB, C_OUT), lambda i: (0, i, 0)),
            pl.BlockSpec((NB, C_OUT), lambda i: (i, 0)),
            pl.BlockSpec((NC, NB, DW), lambda i: (0, i, 0)),
            pl.BlockSpec((1, C_OUT), lambda i: (0, 0)),
        ],
        out_specs=pl.BlockSpec((NB, C_OUT), lambda i: (i, 0)),
        out_shape=jax.ShapeDtypeStruct((N, C_OUT), jnp.float32),
    )(accp, m, degp, gb2)


def kernel(x, edge_index, w1, b1, w2, b2, w3, b3, gw, gb):
    E = edge_index.shape[1]
    ept = -(-E // NW)
    nchunks = -(-ept // CHUNK)
    ept_pad = nchunks * CHUNK
    pad = NW * ept_pad - E

    src = jnp.concatenate([edge_index[0], jnp.zeros((pad,), jnp.int32)])
    dst = jnp.concatenate([edge_index[1], jnp.full((pad,), N, jnp.int32)])

    W1 = jnp.transpose(w1[:, :, 0, :], (2, 1, 0)).reshape(K * C_IN, HID)
    W2 = jnp.transpose(w2[:, :, 0, :], (2, 1, 0)).reshape(K * C_IN, HID)
    W3 = jnp.transpose(w3[:, :, 0, :], (2, 1, 0)).reshape(K * C_IN, HID)
    bstack = jnp.stack([b1, b2, b3, jnp.zeros_like(b1)])

    degp = _make_deg_call(nchunks)(dst)
    degp_n = degp[:, :N, :]

    m = _dense_call(x, W1, W2, W3, bstack, gw, degp_n)

    accp = _make_scatter_call(nchunks)(m, src, dst)

    out = _final_call(accp[:, :N, :], m, degp_n, gb[None, :])
    return out[None]
